# baseline (speedup 1.0000x reference)
import functools

import jax
import jax.numpy as jnp
import numpy as np
from jax import lax
from jax.experimental import pallas as pl
from jax.experimental.pallas import tpu as pltpu
from jax.experimental.pallas import tpu_sc as plsc

PROTO_NUM = 4
N_CLASSES = 1000
FEAT_DIM = 512
BATCH = 16384

L = 16
NC = 2
NS = 16
NW = NC * NS
PER_W = BATCH // NW
CHUNK = 32
NCHUNK = PER_W // CHUNK
GROUP = BATCH // PROTO_NUM
HALF = FEAT_DIM // 2
WPR = HALF // L


def _rsqrt(x):
    i = lax.bitcast_convert_type(x, jnp.int32)
    i = jnp.int32(0x5F3759DF) - lax.shift_right_arithmetic(i, 1)
    y = lax.bitcast_convert_type(i, jnp.float32)
    for _ in range(2):
        y = y * (jnp.float32(1.5) - jnp.float32(0.5) * x * y * y)
    return y


def _sqrt(x):
    return x * _rsqrt(jnp.maximum(x, jnp.float32(1e-35)))


def _body(feat_hbm, y_hbm, table_hbm, out_hbm,
          idx_v, f0, f1, k0, k1, loss_v, sf0, sf1, sk0, sk1):
    cid = lax.axis_index("c")
    sid = lax.axis_index("s")
    wid = sid * NC + cid
    base = wid * PER_W
    goff = (base // GROUP) * N_CLASSES

    pltpu.sync_copy(y_hbm.at[pl.ds(base, PER_W)], idx_v)
    for j in range(PER_W // L):
        sl = pl.ds(j * L, L)
        idx_v[sl] = idx_v[sl] + goff

    fbufs = (f0, f1)
    kbufs = (k0, k1)
    fsems = (sf0, sf1)
    ksems = (sk0, sk1)

    def issue(c):
        b = c % 2
        fcp = pltpu.async_copy(
            feat_hbm.at[pl.ds(base + c * CHUNK, CHUNK)], fbufs[b], fsems[b])
        kcp = pltpu.async_copy(
            table_hbm.at[idx_v.at[pl.ds(c * CHUNK, CHUNK)]], kbufs[b], ksems[b])
        return fcp, kcp

    lane = lax.iota(jnp.int32, L)
    perms = [lax.bitwise_xor(lane, jnp.int32(sh)) for sh in (8, 4, 2, 1)]
    dnums = lax.GatherDimensionNumbers(
        offset_dims=(), collapsed_slice_dims=(0,), start_index_map=(0,))

    def shuffle(x, p):
        return lax.gather(
            x, p[:, None], dnums, (1,),
            mode=lax.GatherScatterMode.PROMISE_IN_BOUNDS)

    def lanesum(x):
        for p in perms:
            x = x + shuffle(x, p)
        return x

    himask = jnp.int32(-65536)

    def compute_chunk(fb, kb, acc):

        def dist(s):
            ff = jnp.zeros((L,), jnp.float32)
            kk = jnp.zeros((L,), jnp.float32)
            fk = jnp.zeros((L,), jnp.float32)
            for j in range(WPR):
                kw = lax.bitcast_convert_type(
                    kb[s, pl.ds(j * L, L)], jnp.int32)
                klo = lax.bitcast_convert_type(
                    lax.shift_left(kw, jnp.int32(16)), jnp.float32)
                khi = lax.bitcast_convert_type(
                    lax.bitwise_and(kw, himask), jnp.float32)
                flo = fb[s, pl.ds(j * L, L)]
                fhi = fb[s, pl.ds(HALF + j * L, L)]
                ff = ff + flo * flo + fhi * fhi
                kk = kk + klo * klo + khi * khi
                fk = fk + flo * klo + fhi * khi
            ffs = lanesum(ff)
            kks = lanesum(kk)
            fks = lanesum(fk)
            inv = _rsqrt(jnp.maximum(ffs * kks, jnp.float32(1e-35)))
            cos = fks * inv
            d2 = jnp.maximum(jnp.float32(2.0) - jnp.float32(2.0) * cos,
                             jnp.float32(0.0))
            return _sqrt(d2)

        def sample(s, a):
            return a + dist(s)

        return lax.fori_loop(0, CHUNK, sample, acc)

    acc = jnp.zeros((L,), jnp.float32)
    pending = issue(0)
    for c in range(NCHUNK):
        fcp, kcp = pending
        fcp.wait()
        kcp.wait()
        if c + 1 < NCHUNK:
            pending = issue(c + 1)
        acc = compute_chunk(fbufs[c % 2], kbufs[c % 2], acc)

    loss_v[...] = acc
    pltpu.sync_copy(loss_v, out_hbm.at[wid])


_PAIR_PERM = np.stack([np.arange(HALF), np.arange(HALF) + HALF], 1).reshape(-1)


@jax.jit
def kernel(feature, y, y_pred, prototype):
    del y_pred
    table = jnp.reshape(prototype, (PROTO_NUM * N_CLASSES, FEAT_DIM))
    table_bf = table.astype(jnp.bfloat16)[:, _PAIR_PERM]
    table_pk = lax.bitcast_convert_type(
        table_bf.reshape(PROTO_NUM * N_CLASSES, HALF, 2), jnp.float32)
    mesh = plsc.VectorSubcoreMesh(core_axis_name="c", subcore_axis_name="s")
    partial = pl.kernel(
        _body,
        out_type=jax.ShapeDtypeStruct((NW, L), jnp.float32),
        mesh=mesh,
        compiler_params=pltpu.CompilerParams(needs_layout_passes=False),
        scratch_types=[
            pltpu.VMEM((PER_W,), jnp.int32),
            pltpu.VMEM((CHUNK, FEAT_DIM), jnp.float32),
            pltpu.VMEM((CHUNK, FEAT_DIM), jnp.float32),
            pltpu.VMEM((CHUNK, HALF), jnp.float32),
            pltpu.VMEM((CHUNK, HALF), jnp.float32),
            pltpu.VMEM((L,), jnp.float32),
            pltpu.SemaphoreType.DMA,
            pltpu.SemaphoreType.DMA,
            pltpu.SemaphoreType.DMA,
            pltpu.SemaphoreType.DMA,
        ],
    )(feature, y, table_pk)
    return jnp.sum(partial) / jnp.float32(L * BATCH)


# speedup vs baseline: 1.7656x; 1.7656x over previous
import functools

import jax
import jax.numpy as jnp
import numpy as np
from jax import lax
from jax.experimental import pallas as pl
from jax.experimental.pallas import tpu as pltpu
from jax.experimental.pallas import tpu_sc as plsc

PROTO_NUM = 4
N_CLASSES = 1000
FEAT_DIM = 512
BATCH = 16384

L = 16
NC = 2
NS = 16
NW = NC * NS
PER_W = BATCH // NW
CHUNK = 32
NCHUNK = PER_W // CHUNK
GROUP = BATCH // PROTO_NUM
HALF = FEAT_DIM // 2
WPR = HALF // L


def _rsqrt(x):
    i = lax.bitcast_convert_type(x, jnp.int32)
    i = jnp.int32(0x5F3759DF) - lax.shift_right_arithmetic(i, 1)
    y = lax.bitcast_convert_type(i, jnp.float32)
    for _ in range(2):
        y = y * (jnp.float32(1.5) - jnp.float32(0.5) * x * y * y)
    return y


def _sqrt(x):
    return x * _rsqrt(jnp.maximum(x, jnp.float32(1e-35)))


def _body(feat_hbm, y_hbm, table_hbm, out_hbm,
          idx_v, f0, f1, k0, k1, loss_v, sf0, sf1, sk0, sk1):
    cid = lax.axis_index("c")
    sid = lax.axis_index("s")
    wid = sid * NC + cid
    base = wid * PER_W
    goff = (base // GROUP) * N_CLASSES

    pltpu.sync_copy(y_hbm.at[pl.ds(base, PER_W)], idx_v)
    for j in range(PER_W // L):
        sl = pl.ds(j * L, L)
        idx_v[sl] = idx_v[sl] + goff

    fbufs = (f0, f1)
    kbufs = (k0, k1)
    fsems = (sf0, sf1)
    ksems = (sk0, sk1)

    def issue(c):
        b = c % 2
        fcp = pltpu.async_copy(
            feat_hbm.at[pl.ds(base + c * CHUNK, CHUNK)], fbufs[b], fsems[b])
        kcp = pltpu.async_copy(
            table_hbm.at[idx_v.at[pl.ds(c * CHUNK, CHUNK)]], kbufs[b], ksems[b])
        return fcp, kcp

    lane = lax.iota(jnp.int32, L)
    perms = [lax.bitwise_xor(lane, jnp.int32(sh)) for sh in (8, 4, 2, 1)]
    dnums = lax.GatherDimensionNumbers(
        offset_dims=(), collapsed_slice_dims=(0,), start_index_map=(0,))

    def shuffle(x, p):
        return lax.gather(
            x, p[:, None], dnums, (1,),
            mode=lax.GatherScatterMode.PROMISE_IN_BOUNDS)

    def lanesum(x):
        for p in perms:
            x = x + shuffle(x, p)
        return x

    himask = jnp.int32(-65536)

    def compute_chunk(fb, kb, acc):

        def dist(s):
            ff = jnp.zeros((L,), jnp.float32)
            kk = jnp.zeros((L,), jnp.float32)
            fk = jnp.zeros((L,), jnp.float32)
            for j in range(WPR):
                kw = lax.bitcast_convert_type(
                    kb[s, pl.ds(j * L, L)], jnp.int32)
                klo = lax.bitcast_convert_type(
                    lax.shift_left(kw, jnp.int32(16)), jnp.float32)
                khi = lax.bitcast_convert_type(
                    lax.bitwise_and(kw, himask), jnp.float32)
                flo = fb[s, pl.ds(j * L, L)]
                fhi = fb[s, pl.ds(HALF + j * L, L)]
                ff = ff + flo * flo + fhi * fhi
                kk = kk + klo * klo + khi * khi
                fk = fk + flo * klo + fhi * khi
            ffs = lanesum(ff)
            kks = lanesum(kk)
            fks = lanesum(fk)
            inv = _rsqrt(jnp.maximum(ffs * kks, jnp.float32(1e-35)))
            cos = fks * inv
            d2 = jnp.maximum(jnp.float32(2.0) - jnp.float32(2.0) * cos,
                             jnp.float32(0.0))
            return _sqrt(d2)

        def sample(s, a):
            return a + dist(s)

        return lax.fori_loop(0, CHUNK, sample, acc)

    acc = jnp.zeros((L,), jnp.float32)
    pending = issue(0)
    for c in range(NCHUNK):
        fcp, kcp = pending
        fcp.wait()
        kcp.wait()
        if c + 1 < NCHUNK:
            pending = issue(c + 1)
        acc = compute_chunk(fbufs[c % 2], kbufs[c % 2], acc)

    loss_v[...] = acc
    pltpu.sync_copy(loss_v, out_hbm.at[wid])


@jax.jit
def kernel(feature, y, y_pred, prototype):
    del y_pred
    table = jnp.reshape(prototype, (PROTO_NUM * N_CLASSES, FEAT_DIM))
    lo = lax.bitcast_convert_type(
        table[:, :HALF].astype(jnp.bfloat16).astype(jnp.float32), jnp.int32)
    hi = lax.bitcast_convert_type(
        table[:, HALF:].astype(jnp.bfloat16).astype(jnp.float32), jnp.int32)
    packed = lax.bitwise_or(
        lax.shift_right_logical(lo, jnp.int32(16)),
        lax.bitwise_and(hi, jnp.int32(-65536)))
    table_pk = lax.bitcast_convert_type(packed, jnp.float32)
    mesh = plsc.VectorSubcoreMesh(core_axis_name="c", subcore_axis_name="s")
    partial = pl.kernel(
        _body,
        out_type=jax.ShapeDtypeStruct((NW, L), jnp.float32),
        mesh=mesh,
        compiler_params=pltpu.CompilerParams(needs_layout_passes=False),
        scratch_types=[
            pltpu.VMEM((PER_W,), jnp.int32),
            pltpu.VMEM((CHUNK, FEAT_DIM), jnp.float32),
            pltpu.VMEM((CHUNK, FEAT_DIM), jnp.float32),
            pltpu.VMEM((CHUNK, HALF), jnp.float32),
            pltpu.VMEM((CHUNK, HALF), jnp.float32),
            pltpu.VMEM((L,), jnp.float32),
            pltpu.SemaphoreType.DMA,
            pltpu.SemaphoreType.DMA,
            pltpu.SemaphoreType.DMA,
            pltpu.SemaphoreType.DMA,
        ],
    )(feature, y, table_pk)
    return jnp.sum(partial) / jnp.float32(L * BATCH)


# speedup vs baseline: 2.0962x; 1.1873x over previous
import functools

import jax
import jax.numpy as jnp
from jax import lax
from jax.experimental import pallas as pl
from jax.experimental.pallas import tpu as pltpu
from jax.experimental.pallas import tpu_sc as plsc

PROTO_NUM = 4
N_CLASSES = 1000
FEAT_DIM = 512
BATCH = 16384
ROWS = PROTO_NUM * N_CLASSES

L = 16
NC = 2
NS = 16
NW = NC * NS
PER_W = BATCH // NW
CHUNK = 32
NCHUNK = PER_W // CHUNK
NBUF = 3
GROUP = BATCH // PROTO_NUM
HALF = FEAT_DIM // 2
WPR = HALF // L
IGM = 128


def _prep_body(table_ref, packed_ref, norms_ref):
    t = table_ref[...]
    tb = t.astype(jnp.bfloat16).astype(jnp.float32)
    norms_ref[...] = jnp.sum(tb * tb, axis=1)
    lo = lax.bitcast_convert_type(tb[:, :HALF], jnp.int32)
    hi = lax.bitcast_convert_type(tb[:, HALF:], jnp.int32)
    packed = lax.bitwise_or(
        lax.shift_right_logical(lo, 16),
        lax.bitwise_and(hi, jnp.int32(-65536)))
    packed_ref[...] = lax.bitcast_convert_type(packed, jnp.float32)


def _rsqrt(x):
    i = lax.bitcast_convert_type(x, jnp.int32)
    i = jnp.int32(0x5F3759DF) - lax.shift_right_arithmetic(i, 1)
    y = lax.bitcast_convert_type(i, jnp.float32)
    for _ in range(2):
        y = y * (jnp.float32(1.5) - jnp.float32(0.5) * x * y * y)
    return y


def _sqrt(x):
    return x * _rsqrt(jnp.maximum(x, jnp.float32(1e-35)))


def _sc_body(feat_hbm, y_hbm, table_hbm, norm_hbm, out_hbm,
             idx_v, norms_v, loss_v, f0, f1, f2, k0, k1, k2,
             sn, sf0, sf1, sf2, sk0, sk1, sk2):
    cid = lax.axis_index("c")
    sid = lax.axis_index("s")
    wid = sid * NC + cid
    base = wid * PER_W
    goff = (base // GROUP) * N_CLASSES

    pltpu.sync_copy(y_hbm.at[pl.ds(base, PER_W)], idx_v)
    for j in range(PER_W // L):
        sl = pl.ds(j * L, L)
        idx_v[sl] = idx_v[sl] + goff

    fbufs = (f0, f1, f2)
    kbufs = (k0, k1, k2)
    fsems = (sf0, sf1, sf2)
    ksems = (sk0, sk1, sk2)

    def issue(c):
        b = c % NBUF
        fcp = pltpu.async_copy(
            feat_hbm.at[pl.ds(base + c * CHUNK, CHUNK)], fbufs[b], fsems[b])
        kcp = pltpu.async_copy(
            table_hbm.at[idx_v.at[pl.ds(c * CHUNK, CHUNK)]], kbufs[b], ksems[b])
        return fcp, kcp

    pend = [issue(0), issue(1)]
    ncps = [
        pltpu.async_copy(
            norm_hbm.at[idx_v.at[pl.ds(i * IGM, IGM)]],
            norms_v.at[pl.ds(i * IGM, IGM)], sn)
        for i in range(PER_W // IGM)
    ]
    for ncp in ncps:
        ncp.wait()

    lane = lax.iota(jnp.int32, L)
    perms = [lax.bitwise_xor(lane, jnp.int32(sh)) for sh in (8, 4, 2, 1)]
    dnums = lax.GatherDimensionNumbers(
        offset_dims=(), collapsed_slice_dims=(0,), start_index_map=(0,))

    def shuffle(x, p):
        return lax.gather(
            x, p[:, None], dnums, (1,),
            mode=lax.GatherScatterMode.PROMISE_IN_BOUNDS)

    def lanesum(x):
        for p in perms:
            x = x + shuffle(x, p)
        return x

    himask = jnp.int32(-65536)

    def compute_chunk(c, acc):
        fb = fbufs[c % NBUF]
        kb = kbufs[c % NBUF]
        nv0 = norms_v[pl.ds(c * CHUNK, L)]
        nv1 = norms_v[pl.ds(c * CHUNK + L, L)]

        def dist(s):
            ff = jnp.zeros((L,), jnp.float32)
            fk = jnp.zeros((L,), jnp.float32)
            for j in range(WPR):
                kw = lax.bitcast_convert_type(
                    kb[s, pl.ds(j * L, L)], jnp.int32)
                klo = lax.bitcast_convert_type(
                    lax.shift_left(kw, jnp.int32(16)), jnp.float32)
                khi = lax.bitcast_convert_type(
                    lax.bitwise_and(kw, himask), jnp.float32)
                flo = fb[s, pl.ds(j * L, L)]
                fhi = fb[s, pl.ds(HALF + j * L, L)]
                ff = ff + flo * flo + fhi * fhi
                fk = fk + flo * klo + fhi * khi
            ffs = lanesum(ff)
            fks = lanesum(fk)
            bs = jnp.broadcast_to(s, (L,)).astype(jnp.int32)
            bs15 = lax.bitwise_and(bs, jnp.int32(L - 1))
            kks = jnp.where(bs < L, shuffle(nv0, bs15), shuffle(nv1, bs15))
            inv = _rsqrt(jnp.maximum(ffs * kks, jnp.float32(1e-35)))
            cos = fks * inv
            d2 = jnp.maximum(jnp.float32(2.0) - jnp.float32(2.0) * cos,
                             jnp.float32(0.0))
            return _sqrt(d2)

        def sample(s, a):
            return a + dist(s)

        return lax.fori_loop(0, CHUNK, sample, acc)

    acc = jnp.zeros((L,), jnp.float32)
    for c in range(NCHUNK):
        fcp, kcp = pend[0]
        fcp.wait()
        kcp.wait()
        pend = pend[1:]
        if c + 2 < NCHUNK:
            pend.append(issue(c + 2))
        acc = compute_chunk(c, acc)

    loss_v[...] = acc
    pltpu.sync_copy(loss_v, out_hbm.at[wid])


@jax.jit
def kernel(feature, y, y_pred, prototype):
    del y_pred
    table = jnp.reshape(prototype, (ROWS, FEAT_DIM))

    table_pk, norms = pl.pallas_call(
        _prep_body,
        out_shape=[
            jax.ShapeDtypeStruct((ROWS, HALF), jnp.float32),
            jax.ShapeDtypeStruct((ROWS,), jnp.float32),
        ],
    )(table)

    mesh = plsc.VectorSubcoreMesh(core_axis_name="c", subcore_axis_name="s")
    partial = pl.kernel(
        _sc_body,
        out_type=jax.ShapeDtypeStruct((NW, L), jnp.float32),
        mesh=mesh,
        compiler_params=pltpu.CompilerParams(needs_layout_passes=False),
        scratch_types=[
            pltpu.VMEM((PER_W,), jnp.int32),
            pltpu.VMEM((PER_W,), jnp.float32),
            pltpu.VMEM((L,), jnp.float32),
            pltpu.VMEM((CHUNK, FEAT_DIM), jnp.float32),
            pltpu.VMEM((CHUNK, FEAT_DIM), jnp.float32),
            pltpu.VMEM((CHUNK, FEAT_DIM), jnp.float32),
            pltpu.VMEM((CHUNK, HALF), jnp.float32),
            pltpu.VMEM((CHUNK, HALF), jnp.float32),
            pltpu.VMEM((CHUNK, HALF), jnp.float32),
            pltpu.SemaphoreType.DMA,
            pltpu.SemaphoreType.DMA,
            pltpu.SemaphoreType.DMA,
            pltpu.SemaphoreType.DMA,
            pltpu.SemaphoreType.DMA,
            pltpu.SemaphoreType.DMA,
            pltpu.SemaphoreType.DMA,
        ],
    )(feature, y, table_pk, norms)
    return jnp.sum(partial) / jnp.float32(L * BATCH)


# speedup vs baseline: 2.1322x; 1.0172x over previous
import functools

import jax
import jax.numpy as jnp
from jax import lax
from jax.experimental import pallas as pl
from jax.experimental.pallas import tpu as pltpu
from jax.experimental.pallas import tpu_sc as plsc

PROTO_NUM = 4
N_CLASSES = 1000
FEAT_DIM = 512
BATCH = 16384
ROWS = PROTO_NUM * N_CLASSES

L = 16
NC = 2
NS = 16
NW = NC * NS
PER_W = BATCH // NW
CHUNK = 32
NCHUNK = PER_W // CHUNK
NBUF = 3
GROUP = BATCH // PROTO_NUM
HALF = FEAT_DIM // 2
WPR = HALF // L


def _prep_body(table_ref, packed_ref):
    t = table_ref[...]
    n = jnp.sqrt(jnp.sum(t * t, axis=1, keepdims=True))
    tn = t / jnp.maximum(n, jnp.float32(1e-12))
    tb = tn.astype(jnp.bfloat16).astype(jnp.float32)
    lo = lax.bitcast_convert_type(tb[:, :HALF], jnp.int32)
    hi = lax.bitcast_convert_type(tb[:, HALF:], jnp.int32)
    packed = lax.bitwise_or(
        lax.shift_right_logical(lo, 16),
        lax.bitwise_and(hi, jnp.int32(-65536)))
    packed_ref[...] = lax.bitcast_convert_type(packed, jnp.float32)


def _rsqrt(x):
    i = lax.bitcast_convert_type(x, jnp.int32)
    i = jnp.int32(0x5F3759DF) - lax.shift_right_arithmetic(i, 1)
    y = lax.bitcast_convert_type(i, jnp.float32)
    for _ in range(2):
        y = y * (jnp.float32(1.5) - jnp.float32(0.5) * x * y * y)
    return y


def _sqrt(x):
    return x * _rsqrt(jnp.maximum(x, jnp.float32(1e-35)))


def _sc_body(feat_hbm, y_hbm, table_hbm, out_hbm,
             idx_v, loss_v, f0, f1, f2, k0, k1, k2,
             sf0, sf1, sf2, sk0, sk1, sk2):
    cid = lax.axis_index("c")
    sid = lax.axis_index("s")
    wid = sid * NC + cid
    base = wid * PER_W
    goff = (base // GROUP) * N_CLASSES

    pltpu.sync_copy(y_hbm.at[pl.ds(base, PER_W)], idx_v)
    for j in range(PER_W // L):
        sl = pl.ds(j * L, L)
        idx_v[sl] = idx_v[sl] + goff

    fbufs = (f0, f1, f2)
    kbufs = (k0, k1, k2)
    fsems = (sf0, sf1, sf2)
    ksems = (sk0, sk1, sk2)

    def issue(c):
        b = c % NBUF
        fcp = pltpu.async_copy(
            feat_hbm.at[pl.ds(base + c * CHUNK, CHUNK)], fbufs[b], fsems[b])
        kcp = pltpu.async_copy(
            table_hbm.at[idx_v.at[pl.ds(c * CHUNK, CHUNK)]], kbufs[b], ksems[b])
        return fcp, kcp

    pend = [issue(0), issue(1)]

    lane = lax.iota(jnp.int32, L)
    perms = [lax.bitwise_xor(lane, jnp.int32(sh)) for sh in (8, 4, 2, 1)]
    dnums = lax.GatherDimensionNumbers(
        offset_dims=(), collapsed_slice_dims=(0,), start_index_map=(0,))

    def shuffle(x, p):
        return lax.gather(
            x, p[:, None], dnums, (1,),
            mode=lax.GatherScatterMode.PROMISE_IN_BOUNDS)

    def lanesum(x):
        for p in perms:
            x = x + shuffle(x, p)
        return x

    himask = jnp.int32(-65536)

    def compute_chunk(c, acc):
        fb = fbufs[c % NBUF]
        kb = kbufs[c % NBUF]

        def dist(s):
            ff = jnp.zeros((L,), jnp.float32)
            fk = jnp.zeros((L,), jnp.float32)
            for j in range(WPR):
                kw = lax.bitcast_convert_type(
                    kb[s, pl.ds(j * L, L)], jnp.int32)
                klo = lax.bitcast_convert_type(
                    lax.shift_left(kw, jnp.int32(16)), jnp.float32)
                khi = lax.bitcast_convert_type(
                    lax.bitwise_and(kw, himask), jnp.float32)
                flo = fb[s, pl.ds(j * L, L)]
                fhi = fb[s, pl.ds(HALF + j * L, L)]
                ff = ff + flo * flo + fhi * fhi
                fk = fk + flo * klo + fhi * khi
            ffs = lanesum(ff)
            fks = lanesum(fk)
            inv = _rsqrt(jnp.maximum(ffs, jnp.float32(1e-35)))
            cos = fks * inv
            d2 = jnp.maximum(jnp.float32(2.0) - jnp.float32(2.0) * cos,
                             jnp.float32(0.0))
            return _sqrt(d2)

        def sample(s, a):
            return a + dist(s)

        return lax.fori_loop(0, CHUNK, sample, acc)

    acc = jnp.zeros((L,), jnp.float32)
    for c in range(NCHUNK):
        fcp, kcp = pend[0]
        fcp.wait()
        kcp.wait()
        pend = pend[1:]
        if c + 2 < NCHUNK:
            pend.append(issue(c + 2))
        acc = compute_chunk(c, acc)

    loss_v[...] = acc
    pltpu.sync_copy(loss_v, out_hbm.at[wid])


@jax.jit
def kernel(feature, y, y_pred, prototype):
    del y_pred
    table = jnp.reshape(prototype, (ROWS, FEAT_DIM))

    prep_rows = ROWS // 10
    table_pk = pl.pallas_call(
        _prep_body,
        grid=(10,),
        in_specs=[pl.BlockSpec((prep_rows, FEAT_DIM), lambda i: (i, 0))],
        out_specs=pl.BlockSpec((prep_rows, HALF), lambda i: (i, 0)),
        out_shape=jax.ShapeDtypeStruct((ROWS, HALF), jnp.float32),
    )(table)

    mesh = plsc.VectorSubcoreMesh(core_axis_name="c", subcore_axis_name="s")
    partial = pl.kernel(
        _sc_body,
        out_type=jax.ShapeDtypeStruct((NW, L), jnp.float32),
        mesh=mesh,
        compiler_params=pltpu.CompilerParams(needs_layout_passes=False),
        scratch_types=[
            pltpu.VMEM((PER_W,), jnp.int32),
            pltpu.VMEM((L,), jnp.float32),
            pltpu.VMEM((CHUNK, FEAT_DIM), jnp.float32),
            pltpu.VMEM((CHUNK, FEAT_DIM), jnp.float32),
            pltpu.VMEM((CHUNK, FEAT_DIM), jnp.float32),
            pltpu.VMEM((CHUNK, HALF), jnp.float32),
            pltpu.VMEM((CHUNK, HALF), jnp.float32),
            pltpu.VMEM((CHUNK, HALF), jnp.float32),
            pltpu.SemaphoreType.DMA,
            pltpu.SemaphoreType.DMA,
            pltpu.SemaphoreType.DMA,
            pltpu.SemaphoreType.DMA,
            pltpu.SemaphoreType.DMA,
            pltpu.SemaphoreType.DMA,
        ],
    )(feature, y, table_pk)
    return jnp.sum(partial) / jnp.float32(L * BATCH)


# speedup vs baseline: 2.2167x; 1.0396x over previous
import functools

import jax
import jax.numpy as jnp
from jax import lax
from jax.experimental import pallas as pl
from jax.experimental.pallas import tpu as pltpu
from jax.experimental.pallas import tpu_sc as plsc

PROTO_NUM = 4
N_CLASSES = 1000
FEAT_DIM = 512
BATCH = 16384
ROWS = PROTO_NUM * N_CLASSES

L = 16
NC = 2
NS = 16
NW = NC * NS
PER_W = BATCH // NW
CHUNK = 32
NCHUNK = PER_W // CHUNK
NBUF = 3
GROUP = BATCH // PROTO_NUM
HALF = FEAT_DIM // 2
WPR = HALF // L


def _prep_body(table_ref, packed_ref):
    t = table_ref[...]
    n = jnp.sqrt(jnp.sum(t * t, axis=1, keepdims=True))
    tn = t / jnp.maximum(n, jnp.float32(1e-12))
    tb = tn.astype(jnp.bfloat16).astype(jnp.float32)
    lo = lax.bitcast_convert_type(tb[:, :HALF], jnp.int32)
    hi = lax.bitcast_convert_type(tb[:, HALF:], jnp.int32)
    packed = lax.bitwise_or(
        lax.shift_right_logical(lo, 16),
        lax.bitwise_and(hi, jnp.int32(-65536)))
    packed_ref[...] = lax.bitcast_convert_type(packed, jnp.float32)


def _rsqrt(x):
    i = lax.bitcast_convert_type(x, jnp.int32)
    i = jnp.int32(0x5F3759DF) - lax.shift_right_arithmetic(i, 1)
    y = lax.bitcast_convert_type(i, jnp.float32)
    for _ in range(2):
        y = y * (jnp.float32(1.5) - jnp.float32(0.5) * x * y * y)
    return y


def _sqrt(x):
    return x * _rsqrt(jnp.maximum(x, jnp.float32(1e-35)))


def _sc_body(feat_hbm, y_hbm, table_hbm, out_hbm,
             idx_v, loss_v, f0, f1, f2, k0, k1, k2,
             sf0, sf1, sf2, sk0, sk1, sk2):
    cid = lax.axis_index("c")
    sid = lax.axis_index("s")
    wid = sid * NC + cid
    base = wid * PER_W
    goff = (base // GROUP) * N_CLASSES

    pltpu.sync_copy(y_hbm.at[pl.ds(base, PER_W)], idx_v)
    for j in range(PER_W // L):
        sl = pl.ds(j * L, L)
        idx_v[sl] = idx_v[sl] + goff

    fbufs = (f0, f1, f2)
    kbufs = (k0, k1, k2)
    fsems = (sf0, sf1, sf2)
    ksems = (sk0, sk1, sk2)

    def issue(c):
        b = c % NBUF
        fcp = pltpu.async_copy(
            feat_hbm.at[pl.ds(base + c * CHUNK, CHUNK)], fbufs[b], fsems[b])
        kcp = pltpu.async_copy(
            table_hbm.at[idx_v.at[pl.ds(c * CHUNK, CHUNK)]], kbufs[b], ksems[b])
        return fcp, kcp

    pend = [issue(0), issue(1)]

    lane = lax.iota(jnp.int32, L)
    perms = [lax.bitwise_xor(lane, jnp.int32(sh)) for sh in (8, 4, 2, 1)]
    dnums = lax.GatherDimensionNumbers(
        offset_dims=(), collapsed_slice_dims=(0,), start_index_map=(0,))

    def shuffle(x, p):
        return lax.gather(
            x, p[:, None], dnums, (1,),
            mode=lax.GatherScatterMode.PROMISE_IN_BOUNDS)

    def lanesum(x):
        for p in perms:
            x = x + shuffle(x, p)
        return x

    himask = jnp.int32(-65536)

    def compute_chunk(c, acc):
        fb = fbufs[c % NBUF]
        kb = kbufs[c % NBUF]

        def dist(s):
            ff = jnp.zeros((L,), jnp.float32)
            fk = jnp.zeros((L,), jnp.float32)
            for j in range(WPR):
                kw = lax.bitcast_convert_type(
                    kb[s, pl.ds(j * L, L)], jnp.int32)
                klo = lax.bitcast_convert_type(
                    lax.shift_left(kw, jnp.int32(16)), jnp.float32)
                khi = lax.bitcast_convert_type(
                    lax.bitwise_and(kw, himask), jnp.float32)
                flo = fb[s, pl.ds(j * L, L)]
                fhi = fb[s, pl.ds(HALF + j * L, L)]
                ff = ff + flo * flo + fhi * fhi
                fk = fk + flo * klo + fhi * khi
            ffs = lanesum(ff)
            fks = lanesum(fk)
            inv = _rsqrt(jnp.maximum(ffs, jnp.float32(1e-35)))
            cos = fks * inv
            d2 = jnp.maximum(jnp.float32(2.0) - jnp.float32(2.0) * cos,
                             jnp.float32(0.0))
            return _sqrt(d2)

        def sample(s, a):
            return a + dist(s)

        return lax.fori_loop(0, CHUNK, sample, acc)

    acc = jnp.zeros((L,), jnp.float32)
    for c in range(NCHUNK):
        fcp, kcp = pend[0]
        fcp.wait()
        kcp.wait()
        pend = pend[1:]
        if c + 2 < NCHUNK:
            pend.append(issue(c + 2))
        acc = compute_chunk(c, acc)

    loss_v[...] = acc
    pltpu.sync_copy(loss_v, out_hbm.at[wid])


@jax.jit
def kernel(feature, y, y_pred, prototype):
    del y_pred
    table = jnp.reshape(prototype, (ROWS, FEAT_DIM))

    prep_rows = ROWS // 4
    table_pk = pl.pallas_call(
        _prep_body,
        grid=(4,),
        in_specs=[pl.BlockSpec((prep_rows, FEAT_DIM), lambda i: (i, 0))],
        out_specs=pl.BlockSpec((prep_rows, HALF), lambda i: (i, 0)),
        out_shape=jax.ShapeDtypeStruct((ROWS, HALF), jnp.float32),
    )(table)

    mesh = plsc.VectorSubcoreMesh(core_axis_name="c", subcore_axis_name="s")
    partial = pl.kernel(
        _sc_body,
        out_type=jax.ShapeDtypeStruct((NW, L), jnp.float32),
        mesh=mesh,
        compiler_params=pltpu.CompilerParams(needs_layout_passes=False),
        scratch_types=[
            pltpu.VMEM((PER_W,), jnp.int32),
            pltpu.VMEM((L,), jnp.float32),
            pltpu.VMEM((CHUNK, FEAT_DIM), jnp.float32),
            pltpu.VMEM((CHUNK, FEAT_DIM), jnp.float32),
            pltpu.VMEM((CHUNK, FEAT_DIM), jnp.float32),
            pltpu.VMEM((CHUNK, HALF), jnp.float32),
            pltpu.VMEM((CHUNK, HALF), jnp.float32),
            pltpu.VMEM((CHUNK, HALF), jnp.float32),
            pltpu.SemaphoreType.DMA,
            pltpu.SemaphoreType.DMA,
            pltpu.SemaphoreType.DMA,
            pltpu.SemaphoreType.DMA,
            pltpu.SemaphoreType.DMA,
            pltpu.SemaphoreType.DMA,
        ],
    )(feature, y, table_pk)
    return jnp.sum(partial) / jnp.float32(L * BATCH)


# speedup vs baseline: 2.2223x; 1.0025x over previous
import functools

import jax
import jax.numpy as jnp
from jax import lax
from jax.experimental import pallas as pl
from jax.experimental.pallas import tpu as pltpu
from jax.experimental.pallas import tpu_sc as plsc

PROTO_NUM = 4
N_CLASSES = 1000
FEAT_DIM = 512
BATCH = 16384
ROWS = PROTO_NUM * N_CLASSES

L = 16
NC = 2
NS = 16
NW = NC * NS
PER_W = BATCH // NW
CHUNK = 32
NCHUNK = PER_W // CHUNK
NBUF = 3
GROUP = BATCH // PROTO_NUM
HALF = FEAT_DIM // 2
WPR = HALF // L


def _prep_body(table_ref, packed_ref):
    t = table_ref[...]
    n = jnp.sqrt(jnp.sum(t * t, axis=1, keepdims=True))
    tn = t / jnp.maximum(n, jnp.float32(1e-12))
    tb = tn.astype(jnp.bfloat16).astype(jnp.float32)
    lo = lax.bitcast_convert_type(tb[:, :HALF], jnp.int32)
    hi = lax.bitcast_convert_type(tb[:, HALF:], jnp.int32)
    packed = lax.bitwise_or(
        lax.shift_right_logical(lo, 16),
        lax.bitwise_and(hi, jnp.int32(-65536)))
    packed_ref[...] = lax.bitcast_convert_type(packed, jnp.float32)


def _rsqrt(x):
    i = lax.bitcast_convert_type(x, jnp.int32)
    i = jnp.int32(0x5F3759DF) - lax.shift_right_arithmetic(i, 1)
    y = lax.bitcast_convert_type(i, jnp.float32)
    for _ in range(2):
        y = y * (jnp.float32(1.5) - jnp.float32(0.5) * x * y * y)
    return y


def _sqrt(x):
    return x * _rsqrt(jnp.maximum(x, jnp.float32(1e-35)))


def _sc_body(feat_hbm, y_hbm, table_hbm, out_hbm,
             idx_v, loss_v, f0, f1, f2, k0, k1, k2,
             sf0, sf1, sf2, sk0, sk1, sk2):
    cid = lax.axis_index("c")
    sid = lax.axis_index("s")
    wid = sid * NC + cid
    base = wid * PER_W
    goff = (base // GROUP) * N_CLASSES

    pltpu.sync_copy(y_hbm.at[pl.ds(base, PER_W)], idx_v)
    for j in range(PER_W // L):
        sl = pl.ds(j * L, L)
        idx_v[sl] = idx_v[sl] + goff

    fbufs = (f0, f1, f2)
    kbufs = (k0, k1, k2)
    fsems = (sf0, sf1, sf2)
    ksems = (sk0, sk1, sk2)

    def issue(c):
        b = c % NBUF
        fcp = pltpu.async_copy(
            feat_hbm.at[pl.ds(base + c * CHUNK, CHUNK)], fbufs[b], fsems[b])
        kcp = pltpu.async_copy(
            table_hbm.at[idx_v.at[pl.ds(c * CHUNK, CHUNK)]], kbufs[b], ksems[b])
        return fcp, kcp

    pend = [issue(0), issue(1)]

    lane = lax.iota(jnp.int32, L)
    perms = [lax.bitwise_xor(lane, jnp.int32(sh)) for sh in (8, 4, 2, 1)]
    dnums = lax.GatherDimensionNumbers(
        offset_dims=(), collapsed_slice_dims=(0,), start_index_map=(0,))

    def shuffle(x, p):
        return lax.gather(
            x, p[:, None], dnums, (1,),
            mode=lax.GatherScatterMode.PROMISE_IN_BOUNDS)

    def lanesum(x):
        for p in perms:
            x = x + shuffle(x, p)
        return x

    himask = jnp.int32(-65536)

    def compute_chunk(c, acc):
        fb = fbufs[c % NBUF]
        kb = kbufs[c % NBUF]

        def dist(s):
            ff = jnp.zeros((L,), jnp.float32)
            fk = jnp.zeros((L,), jnp.float32)
            for j in range(WPR):
                kw = lax.bitcast_convert_type(
                    kb[s, pl.ds(j * L, L)], jnp.int32)
                klo = lax.bitcast_convert_type(
                    lax.shift_left(kw, jnp.int32(16)), jnp.float32)
                khi = lax.bitcast_convert_type(
                    lax.bitwise_and(kw, himask), jnp.float32)
                flo = fb[s, pl.ds(j * L, L)]
                fhi = fb[s, pl.ds(HALF + j * L, L)]
                ff = ff + flo * flo + fhi * fhi
                fk = fk + flo * klo + fhi * khi
            ffs = lanesum(ff)
            fks = lanesum(fk)
            inv = _rsqrt(jnp.maximum(ffs, jnp.float32(1e-35)))
            cos = fks * inv
            d2 = jnp.maximum(jnp.float32(2.0) - jnp.float32(2.0) * cos,
                             jnp.float32(0.0))
            return _sqrt(d2)

        def sample(s, a):
            return a + dist(s)

        return lax.fori_loop(0, CHUNK, sample, acc)

    acc = jnp.zeros((L,), jnp.float32)
    for c in range(NCHUNK):
        fcp, kcp = pend[0]
        fcp.wait()
        kcp.wait()
        pend = pend[1:]
        if c + 2 < NCHUNK:
            pend.append(issue(c + 2))
        acc = compute_chunk(c, acc)

    loss_v[...] = acc
    pltpu.sync_copy(loss_v, out_hbm.at[wid])


@jax.jit
def kernel(feature, y, y_pred, prototype):
    del y_pred
    table = jnp.reshape(prototype, (ROWS, FEAT_DIM))

    prep_rows = ROWS // 2
    table_pk = pl.pallas_call(
        _prep_body,
        grid=(2,),
        in_specs=[pl.BlockSpec((prep_rows, FEAT_DIM), lambda i: (i, 0))],
        out_specs=pl.BlockSpec((prep_rows, HALF), lambda i: (i, 0)),
        out_shape=jax.ShapeDtypeStruct((ROWS, HALF), jnp.float32),
    )(table)

    mesh = plsc.VectorSubcoreMesh(core_axis_name="c", subcore_axis_name="s")
    partial = pl.kernel(
        _sc_body,
        out_type=jax.ShapeDtypeStruct((NW, L), jnp.float32),
        mesh=mesh,
        compiler_params=pltpu.CompilerParams(needs_layout_passes=False),
        scratch_types=[
            pltpu.VMEM((PER_W,), jnp.int32),
            pltpu.VMEM((L,), jnp.float32),
            pltpu.VMEM((CHUNK, FEAT_DIM), jnp.float32),
            pltpu.VMEM((CHUNK, FEAT_DIM), jnp.float32),
            pltpu.VMEM((CHUNK, FEAT_DIM), jnp.float32),
            pltpu.VMEM((CHUNK, HALF), jnp.float32),
            pltpu.VMEM((CHUNK, HALF), jnp.float32),
            pltpu.VMEM((CHUNK, HALF), jnp.float32),
            pltpu.SemaphoreType.DMA,
            pltpu.SemaphoreType.DMA,
            pltpu.SemaphoreType.DMA,
            pltpu.SemaphoreType.DMA,
            pltpu.SemaphoreType.DMA,
            pltpu.SemaphoreType.DMA,
        ],
    )(feature, y, table_pk)
    return jnp.sum(partial) / jnp.float32(L * BATCH)
